# baseline (speedup 1.0000x reference)
import functools

import jax
import jax.numpy as jnp
from jax import lax
from jax.experimental import pallas as pl
from jax.experimental.pallas import tpu as pltpu
from jax.experimental.pallas import tpu_sc as plsc

B = 4096
NC = 64
NF = 64
N2 = NC + NF
NEAR = 2.0
FAR = 6.0
STEP = (FAR - NEAR) / (NC - 1)
K = 8
V = 128
MINB = -4.0
MAXB = 4.0
VD = 16
HID = 64
T = 100
GSCALE = (V - 1) / (MAXB - MINB)

SC_CORES = 2
SC_SUBCORES = 16
NW = SC_CORES * SC_SUBCORES

HIGH = lax.Precision.HIGHEST


def _sum0(x):
    while x.shape[0] > 1:
        s = x.shape[0]
        h = s // 2
        y = x[:h] + x[h:2 * h]
        x = y if s % 2 == 0 else jnp.concatenate([y, x[2 * h:]], axis=0)
    return x[0]


def _max0(x):
    while x.shape[0] > 1:
        s = x.shape[0]
        h = s // 2
        y = jnp.maximum(x[:h], x[h:2 * h])
        x = y if s % 2 == 0 else jnp.concatenate([y, x[2 * h:]], axis=0)
    return x[0]


def _min0(x):
    while x.shape[0] > 1:
        s = x.shape[0]
        h = s // 2
        y = jnp.minimum(x[:h], x[h:2 * h])
        x = y if s % 2 == 0 else jnp.concatenate([y, x[2 * h:]], axis=0)
    return x[0]


def _bt_t(ts_row, mbt_ref):
    r = ts_row.shape[1]
    onehot = (lax.broadcasted_iota(jnp.int32, (T, r), 0)
              == ts_row).astype(jnp.float32)
    return lax.dot_general(mbt_ref[...], onehot, (((1,), (0,)), ((), ())),
                           precision=HIGH)


def _warp_corners_t(oc, dc, depths_t, bt_t, mw1t_ref, mb1c_ref, mw2_ref,
                    mb2r_ref):
    w1 = mw1t_ref[...]
    w2 = mw2_ref[...]
    mb2r = mb2r_ref[...]
    w1c = [w1[:, c:c + 1] for c in range(3)]
    a2 = (w1c[0] * oc[0] + w1c[1] * oc[1] + w1c[2] * oc[2]
          + mb1c_ref[...])
    b2 = w1c[0] * dc[0] + w1c[1] * dc[1] + w1c[2] * dc[2]
    h = jnp.tanh(a2[:, None, :] + b2[:, None, :] * depths_t[None, :, :])

    gs = []
    for c in range(3):
        vc = w2[:, c:c + 1] * bt_t[0:1, :]
        mbc = mb2r[0:1, c:c + 1] * bt_t[0:1, :]
        for k in range(1, K):
            col = 3 * k + c
            vc = vc + w2[:, col:col + 1] * bt_t[k:k + 1, :]
            mbc = mbc + mb2r[0:1, col:col + 1] * bt_t[k:k + 1, :]
        disp = _sum0(h * vc[:, None, :]) + mbc
        wpos = oc[c] + dc[c] * depths_t + disp
        g = (wpos - MINB) * GSCALE
        gs.append(jnp.clip(g, 0.0, V - 1 - 1e-6))

    g0f = [jnp.floor(g) for g in gs]
    f = [g - g0 for g, g0 in zip(gs, g0f)]
    gi = [g0.astype(jnp.int32) for g0 in g0f]
    hi = [jnp.minimum(g + 1, V - 1) for g in gi]
    gx, gy, gz = gi
    hx, hy, hz = hi
    fx, fy, fz = f
    ex, ey, ez = 1.0 - fx, 1.0 - fy, 1.0 - fz
    idx8 = [(a_ * V + b_) * V + c_ for a_, b_, c_ in
            ((gx, gy, gz), (gx, gy, hz), (gx, hy, gz), (gx, hy, hz),
             (hx, gy, gz), (hx, gy, hz), (hx, hy, gz), (hx, hy, hz))]
    w8 = [ex * ey * ez, ex * ey * fz, ex * fy * ez, ex * fy * fz,
          fx * ey * ez, fx * ey * fz, fx * fy * ez, fx * fy * fz]
    return idx8, w8


def _prep_write(ors, drs, ts_ref, mbt_ref, mw1t_ref, mb1c_ref, mw2_ref,
                mb2r_ref, depths_t, idx_ref, w_ref):
    bt_t = _bt_t(ts_ref[...], mbt_ref)
    oc = [o_ref[...] for o_ref in ors]
    dc = [d_ref[...] for d_ref in drs]
    idx8, w8 = _warp_corners_t(oc, dc, depths_t, bt_t, mw1t_ref, mb1c_ref,
                               mw2_ref, mb2r_ref)
    for j in range(8):
        idx_ref[j] = idx8[j]
        w_ref[j] = w8[j]


def _tc1_body(ox, oy, oz, dx, dy, dz, ts_ref, mbt_ref, mw1t_ref, mb1c_ref,
              mw2_ref, mb2r_ref, idx_ref, w_ref):
    r = ox.shape[1]
    depths_t = NEAR + STEP * lax.broadcasted_iota(
        jnp.int32, (NC, r), 0).astype(jnp.float32)
    _prep_write((ox, oy, oz), (dx, dy, dz), ts_ref, mbt_ref, mw1t_ref,
                mb1c_ref, mw2_ref, mb2r_ref, depths_t, idx_ref, w_ref)


def _row_spec(r):
    return pl.BlockSpec((1, r), lambda i: (0, i))


def _coarse_prep(rows, ts_row, mbt, mw1t, mb1c, mw2, mb2r):
    r = 512
    grid = B // r
    return pl.pallas_call(
        _tc1_body,
        grid=(grid,),
        in_specs=[_row_spec(r)] * 6 + [
            pl.BlockSpec((1, r), lambda i: (0, i)),
            pl.BlockSpec((K, T), lambda i: (0, 0)),
            pl.BlockSpec((HID, 3), lambda i: (0, 0)),
            pl.BlockSpec((HID, 1), lambda i: (0, 0)),
            pl.BlockSpec((HID, 3 * K), lambda i: (0, 0)),
            pl.BlockSpec((1, 3 * K), lambda i: (0, 0)),
        ],
        out_specs=[
            pl.BlockSpec((8, NC, r), lambda i: (0, 0, i)),
            pl.BlockSpec((8, NC, r), lambda i: (0, 0, i)),
        ],
        out_shape=[
            jax.ShapeDtypeStruct((8, NC, B), jnp.int32),
            jax.ShapeDtypeStruct((8, NC, B), jnp.float32),
        ],
    )(*rows, ts_row, mbt, mw1t, mb1c, mw2, mb2r)


def _make_gather_blend(p_total, chunk):
    pw = p_total // NW
    nch = pw // chunk
    assert pw % chunk == 0 and chunk % 128 == 0 and nch % 2 == 0
    mesh = plsc.VectorSubcoreMesh(core_axis_name="c", subcore_axis_name="s",
                                  num_cores=SC_CORES, num_subcores=SC_SUBCORES)

    @functools.partial(
        pl.kernel,
        out_type=jax.ShapeDtypeStruct((p_total, VD), jnp.float32),
        mesh=mesh,
        scratch_types=[
            pltpu.VMEM((8 * chunk,), jnp.int32),
            pltpu.VMEM((8 * chunk,), jnp.int32),
            pltpu.VMEM((8 * chunk, VD), jnp.float32),
            pltpu.VMEM((8 * chunk, VD), jnp.float32),
            pltpu.VMEM((8 * chunk,), jnp.float32),
            pltpu.VMEM((8 * chunk,), jnp.float32),
            pltpu.VMEM((chunk, VD), jnp.float32),
            pltpu.SemaphoreType.DMA,
            pltpu.SemaphoreType.DMA,
        ],
        compiler_params=pltpu.CompilerParams(use_tc_tiling_on_sc=False,
                                             needs_layout_passes=False),
    )
    def k(table, idx2d, w2d, out_hbm, idx0, idx1, rows0, rows1, w0, w1,
          out_v, sem0, sem1):
        cid = lax.axis_index("c")
        sid = lax.axis_index("s")
        wid = sid * SC_CORES + cid
        idx_b = (idx0, idx1)
        rows_b = (rows0, rows1)
        w_b = (w0, w1)
        sem_b = (sem0, sem1)
        lanes = lax.iota(jnp.int32, 16)

        def stage(ch, buf):
            base = wid * pw + ch * chunk
            for j in range(8):
                pltpu.sync_copy(idx2d.at[j, pl.ds(base, chunk)],
                                idx_b[buf].at[pl.ds(j * chunk, chunk)])
                pltpu.sync_copy(w2d.at[j, pl.ds(base, chunk)],
                                w_b[buf].at[pl.ds(j * chunk, chunk)])
            pltpu.async_copy(table.at[idx_b[buf]], rows_b[buf], sem_b[buf])

        def blend(ch, buf):
            base = wid * pw + ch * chunk
            pltpu.make_async_copy(
                table.at[idx_b[buf]], rows_b[buf], sem_b[buf]).wait()

            def tile_body(tb, c2):
                p16 = tb * 16 + lanes
                wv = [w_b[buf][pl.ds(j * chunk + tb * 16, 16)]
                      for j in range(8)]
                for c in range(VD):
                    csplat = jnp.full((16,), c, jnp.int32)
                    acc = jnp.zeros((16,), jnp.float32)
                    for j in range(8):
                        g = plsc.load_gather(
                            rows_b[buf], [p16 + j * chunk, csplat])
                        acc = acc + wv[j] * g
                    plsc.store_scatter(out_v, [p16, csplat], acc)
                return c2

            lax.fori_loop(0, chunk // 16, tile_body, 0)
            pltpu.sync_copy(out_v, out_hbm.at[pl.ds(base, chunk), :])

        stage(0, 0)

        def pair_body(g, carry):
            ch0 = g * 2
            stage(ch0 + 1, 1)
            blend(ch0, 0)

            @pl.when(ch0 + 2 < nch)
            def _():
                stage(ch0 + 2, 0)

            blend(ch0 + 1, 1)
            return carry

        lax.fori_loop(0, nch // 2, pair_body, 0)

    return k


def _tc2_body(feat0_ref, ox, oy, oz, dx, dy, dz, ts_ref, mbt_ref, mw1t_ref,
              mb1c_ref, mw2_ref, mb2r_ref, u_ref, dv2_ref, idx_ref, w_ref):
    r = ox.shape[1]
    feat0 = feat0_ref[...]
    dens = jnp.maximum(feat0, 0.0) + jnp.log1p(jnp.exp(-jnp.abs(feat0)))

    dvv = NEAR + STEP * lax.broadcasted_iota(
        jnp.int32, (NC, 1), 0).astype(jnp.float32)
    dist = jnp.concatenate(
        [dvv[1:] - dvv[:-1], jnp.full((1, 1), 1e10, jnp.float32)], axis=0)
    e = jnp.exp(-dens * dist)
    alpha = 1.0 - e
    vte = e + 1e-10
    logv = jnp.log(vte)
    mstrict = (lax.broadcasted_iota(jnp.int32, (NC, NC), 1)
               < lax.broadcasted_iota(jnp.int32, (NC, NC), 0)).astype(jnp.float32)
    tt = jnp.exp(lax.dot_general(mstrict, logv, (((1,), (0,)), ((), ())),
                                 precision=HIGH))
    w = tt * alpha

    m = NC - 2
    wp = w[1:NC - 1, :] + 1e-5
    ones = jnp.ones((1, m), jnp.float32)
    wsum = lax.dot_general(ones, wp, (((1,), (0,)), ((), ())), precision=HIGH)
    pdf = wp / wsum
    minc = (lax.broadcasted_iota(jnp.int32, (m, m), 1)
            <= lax.broadcasted_iota(jnp.int32, (m, m), 0)).astype(jnp.float32)
    cdf = lax.dot_general(minc, pdf, (((1,), (0,)), ((), ())), precision=HIGH)
    c62 = jnp.concatenate([jnp.zeros((1, r), jnp.float32), cdf[:m - 1, :]],
                          axis=0)
    u = u_ref[...]
    ge = u[None, :, :] >= c62[:, None, :]
    gef = ge.astype(jnp.float32)
    above_f = _sum0(gef)
    cdf_b = _max0(gef * c62[:, None, :])
    cdf_a3 = jnp.where(ge, 2.0, jnp.broadcast_to(c62[:, None, :], ge.shape))
    cdf_a = jnp.minimum(_min0(cdf_a3), cdf[m - 1:m, :])
    denom = cdf_a - cdf_b
    denom = jnp.where(denom < 1e-5, 1.0, denom)
    t = (u - cdf_b) / denom
    bins_b = NEAR + STEP * (above_f - 0.5)
    samples = bins_b + t * STEP

    x = jnp.concatenate([jnp.broadcast_to(dvv, (NC, r)), samples], axis=0)
    sub = lax.broadcasted_iota(jnp.int32, (N2, 1), 0)
    k = 2
    while k <= N2:
        asc = (sub & k) == 0
        j = k // 2
        while j >= 1:
            upper = (sub & j) != 0
            partner = jnp.where(upper, jnp.roll(x, j, axis=0),
                                jnp.roll(x, -j, axis=0))
            takemin = (~upper) == asc
            x = jnp.where(takemin, jnp.minimum(x, partner),
                          jnp.maximum(x, partner))
            j //= 2
        k *= 2
    dv2_ref[...] = x
    _prep_write((ox, oy, oz), (dx, dy, dz), ts_ref, mbt_ref, mw1t_ref,
                mb1c_ref, mw2_ref, mb2r_ref, x, idx_ref, w_ref)


def _fine_prep(feat0c, rows, ts_row, mbt, mw1t, mb1c, mw2, mb2r, u_t):
    r = 256
    grid = B // r
    return pl.pallas_call(
        _tc2_body,
        grid=(grid,),
        in_specs=[pl.BlockSpec((NC, r), lambda i: (0, i))]
        + [_row_spec(r)] * 6 + [
            pl.BlockSpec((1, r), lambda i: (0, i)),
            pl.BlockSpec((K, T), lambda i: (0, 0)),
            pl.BlockSpec((HID, 3), lambda i: (0, 0)),
            pl.BlockSpec((HID, 1), lambda i: (0, 0)),
            pl.BlockSpec((HID, 3 * K), lambda i: (0, 0)),
            pl.BlockSpec((1, 3 * K), lambda i: (0, 0)),
            pl.BlockSpec((NF, r), lambda i: (0, i)),
        ],
        out_specs=[
            pl.BlockSpec((N2, r), lambda i: (0, i)),
            pl.BlockSpec((8, N2, r), lambda i: (0, 0, i)),
            pl.BlockSpec((8, N2, r), lambda i: (0, 0, i)),
        ],
        out_shape=[
            jax.ShapeDtypeStruct((N2, B), jnp.float32),
            jax.ShapeDtypeStruct((8, N2, B), jnp.int32),
            jax.ShapeDtypeStruct((8, N2, B), jnp.float32),
        ],
    )(feat0c, *rows, ts_row, mbt, mw1t, mb1c, mw2, mb2r, u_t)


def _tc3a_body(feat_ref, cw1_ref, cb1_ref, cw2_ref, cb2_ref, out_ref):
    f2 = feat_ref[...]
    h = jnp.maximum(jnp.dot(f2, cw1_ref[...]) + cb1_ref[...], 0.0)
    out_ref[...] = jnp.dot(h, cw2_ref[...]) + cb2_ref[...]


def _color_mlp(featf, cw1, cb1, cw2, cb2):
    rp = 8192
    grid = (B * N2) // rp
    return pl.pallas_call(
        _tc3a_body,
        grid=(grid,),
        in_specs=[
            pl.BlockSpec((rp, VD), lambda i: (i, 0)),
            pl.BlockSpec((VD, HID), lambda i: (0, 0)),
            pl.BlockSpec((1, HID), lambda i: (0, 0)),
            pl.BlockSpec((HID, 3), lambda i: (0, 0)),
            pl.BlockSpec((1, 3), lambda i: (0, 0)),
        ],
        out_specs=pl.BlockSpec((rp, 3), lambda i: (i, 0)),
        out_shape=jax.ShapeDtypeStruct((B * N2, 3), jnp.float32),
    )(featf, cw1, cb1, cw2, cb2)


def _tc3b_body(feat0_ref, lx_ref, ly_ref, lz_ref, dv2_ref, out_ref):
    feat0 = feat0_ref[...]
    dens = jnp.maximum(feat0, 0.0) + jnp.log1p(jnp.exp(-jnp.abs(feat0)))
    dv2 = dv2_ref[...]
    r = dv2.shape[1]
    dist = jnp.concatenate(
        [dv2[1:, :] - dv2[:-1, :], jnp.full((1, r), 1e10, jnp.float32)], axis=0)
    e = jnp.exp(-dens * dist)
    alpha = 1.0 - e
    vte = e + 1e-10
    logv = jnp.log(vte)
    mstrict = (lax.broadcasted_iota(jnp.int32, (N2, N2), 1)
               < lax.broadcasted_iota(jnp.int32, (N2, N2), 0)).astype(jnp.float32)
    tt = jnp.exp(lax.dot_general(mstrict, logv, (((1,), (0,)), ((), ())),
                                 precision=HIGH))
    w = tt * alpha
    ones = jnp.ones((1, N2), jnp.float32)
    chans = [lax.dot_general(ones, w * jax.nn.sigmoid(c_ref[...]),
                             (((1,), (0,)), ((), ())), precision=HIGH)
             for c_ref in (lx_ref, ly_ref, lz_ref)]
    out_ref[...] = jnp.concatenate(chans, axis=0)


def _final_render(feat0f, lx, ly, lz, dv2):
    r = 512
    grid = B // r
    return pl.pallas_call(
        _tc3b_body,
        grid=(grid,),
        in_specs=[pl.BlockSpec((N2, r), lambda i: (0, i))] * 5,
        out_specs=pl.BlockSpec((3, r), lambda i: (0, i)),
        out_shape=jax.ShapeDtypeStruct((3, B), jnp.float32),
    )(feat0f, lx, ly, lz, dv2)


def kernel(ray_origin, ray_direction, time_step, motion_basis, mw1, mb1, mw2,
           mb2, voxel_grid, cw1, cb1, cw2, cb2):
    ot = ray_origin.T
    dt = ray_direction.T
    ts_row = time_step.astype(jnp.int32).reshape(1, B)
    rows = (ot[0:1], ot[1:2], ot[2:3], dt[0:1], dt[1:2], dt[2:3])
    mbt = motion_basis.T
    mw1t = mw1.T
    mb1c = mb1.reshape(HID, 1)
    mb2r = mb2.reshape(1, 3 * K)
    cb1r = cb1.reshape(1, HID)
    table = voxel_grid.reshape(V * V * V, VD)
    u_t = jax.random.uniform(jax.random.key(1), (B, NF), dtype=jnp.float32).T

    idxc, wc = _coarse_prep(rows, ts_row, mbt, mw1t, mb1c, mw2, mb2r)
    pc = B * NC
    featc = _make_gather_blend(pc, 256)(
        table, idxc.reshape(8, pc), wc.reshape(8, pc))
    feat0c = featc[:, 0].reshape(NC, B)
    dv2, idxf, wf = _fine_prep(feat0c, rows, ts_row, mbt, mw1t, mb1c, mw2,
                               mb2r, u_t)
    pf = B * N2
    featf = _make_gather_blend(pf, 256)(
        table, idxf.reshape(8, pf), wf.reshape(8, pf))
    logits = _color_mlp(featf, cw1, cb1r, cw2, cb2.reshape(1, 3))
    l3 = logits.reshape(N2, B, 3)
    feat0f = featf[:, 0].reshape(N2, B)
    colors_t = _final_render(feat0f, l3[:, :, 0], l3[:, :, 1], l3[:, :, 2],
                             dv2)
    return colors_t.T


# speedup vs baseline: 1.1920x; 1.1920x over previous
import functools

import jax
import jax.numpy as jnp
from jax import lax
from jax.experimental import pallas as pl
from jax.experimental.pallas import tpu as pltpu
from jax.experimental.pallas import tpu_sc as plsc

B = 4096
NC = 64
NF = 64
N2 = NC + NF
NEAR = 2.0
FAR = 6.0
STEP = (FAR - NEAR) / (NC - 1)
K = 8
V = 128
MINB = -4.0
MAXB = 4.0
VD = 16
HID = 64
T = 100
GSCALE = (V - 1) / (MAXB - MINB)

SC_CORES = 2
SC_SUBCORES = 16
NW = SC_CORES * SC_SUBCORES

HIGH = lax.Precision.HIGHEST


def _sum0(x):
    while x.shape[0] > 1:
        s = x.shape[0]
        h = s // 2
        y = x[:h] + x[h:2 * h]
        x = y if s % 2 == 0 else jnp.concatenate([y, x[2 * h:]], axis=0)
    return x[0]


def _max0(x):
    while x.shape[0] > 1:
        s = x.shape[0]
        h = s // 2
        y = jnp.maximum(x[:h], x[h:2 * h])
        x = y if s % 2 == 0 else jnp.concatenate([y, x[2 * h:]], axis=0)
    return x[0]


def _min0(x):
    while x.shape[0] > 1:
        s = x.shape[0]
        h = s // 2
        y = jnp.minimum(x[:h], x[h:2 * h])
        x = y if s % 2 == 0 else jnp.concatenate([y, x[2 * h:]], axis=0)
    return x[0]


def _bt_t(ts_row, mbt_ref):
    r = ts_row.shape[1]
    onehot = (lax.broadcasted_iota(jnp.int32, (T, r), 0)
              == ts_row).astype(jnp.float32)
    return lax.dot_general(mbt_ref[...], onehot, (((1,), (0,)), ((), ())),
                           precision=HIGH)


def _warp_corners_t(oc, dc, depths_t, bt_t, mw1t_ref, mb1c_ref, mw2_ref,
                    mb2r_ref):
    w1 = mw1t_ref[...]
    w2 = mw2_ref[...]
    mb2r = mb2r_ref[...]
    w1c = [w1[:, c:c + 1] for c in range(3)]
    a2 = (w1c[0] * oc[0] + w1c[1] * oc[1] + w1c[2] * oc[2]
          + mb1c_ref[...])
    b2 = w1c[0] * dc[0] + w1c[1] * dc[1] + w1c[2] * dc[2]
    h = jnp.tanh(a2[:, None, :] + b2[:, None, :] * depths_t[None, :, :])

    gs = []
    for c in range(3):
        vc = w2[:, c:c + 1] * bt_t[0:1, :]
        mbc = mb2r[0:1, c:c + 1] * bt_t[0:1, :]
        for k in range(1, K):
            col = 3 * k + c
            vc = vc + w2[:, col:col + 1] * bt_t[k:k + 1, :]
            mbc = mbc + mb2r[0:1, col:col + 1] * bt_t[k:k + 1, :]
        disp = _sum0(h * vc[:, None, :]) + mbc
        wpos = oc[c] + dc[c] * depths_t + disp
        g = (wpos - MINB) * GSCALE
        gs.append(jnp.clip(g, 0.0, V - 1 - 1e-6))

    g0f = [jnp.floor(g) for g in gs]
    f = [g - g0 for g, g0 in zip(gs, g0f)]
    gi = [g0.astype(jnp.int32) for g0 in g0f]
    hi = [jnp.minimum(g + 1, V - 1) for g in gi]
    gx, gy, gz = gi
    hx, hy, hz = hi
    fx, fy, fz = f
    ex, ey, ez = 1.0 - fx, 1.0 - fy, 1.0 - fz
    idx8 = [(a_ * V + b_) * V + c_ for a_, b_, c_ in
            ((gx, gy, gz), (gx, gy, hz), (gx, hy, gz), (gx, hy, hz),
             (hx, gy, gz), (hx, gy, hz), (hx, hy, gz), (hx, hy, hz))]
    w8 = [ex * ey * ez, ex * ey * fz, ex * fy * ez, ex * fy * fz,
          fx * ey * ez, fx * ey * fz, fx * fy * ez, fx * fy * fz]
    return idx8, w8


def _prep_write(ors, drs, ts_ref, mbt_ref, mw1t_ref, mb1c_ref, mw2_ref,
                mb2r_ref, depths_t, idx_ref, w_ref):
    bt_t = _bt_t(ts_ref[...], mbt_ref)
    oc = [o_ref[...] for o_ref in ors]
    dc = [d_ref[...] for d_ref in drs]
    idx8, w8 = _warp_corners_t(oc, dc, depths_t, bt_t, mw1t_ref, mb1c_ref,
                               mw2_ref, mb2r_ref)
    for j in range(8):
        idx_ref[j] = idx8[j]
        w_ref[j] = w8[j]


def _tc1_body(ox, oy, oz, dx, dy, dz, ts_ref, mbt_ref, mw1t_ref, mb1c_ref,
              mw2_ref, mb2r_ref, idx_ref, w_ref):
    r = ox.shape[1]
    depths_t = NEAR + STEP * lax.broadcasted_iota(
        jnp.int32, (NC, r), 0).astype(jnp.float32)
    _prep_write((ox, oy, oz), (dx, dy, dz), ts_ref, mbt_ref, mw1t_ref,
                mb1c_ref, mw2_ref, mb2r_ref, depths_t, idx_ref, w_ref)


def _row_spec(r):
    return pl.BlockSpec((1, r), lambda i: (0, i))


def _coarse_prep(rows, ts_row, mbt, mw1t, mb1c, mw2, mb2r):
    r = 512
    grid = B // r
    return pl.pallas_call(
        _tc1_body,
        grid=(grid,),
        in_specs=[_row_spec(r)] * 6 + [
            pl.BlockSpec((1, r), lambda i: (0, i)),
            pl.BlockSpec((K, T), lambda i: (0, 0)),
            pl.BlockSpec((HID, 3), lambda i: (0, 0)),
            pl.BlockSpec((HID, 1), lambda i: (0, 0)),
            pl.BlockSpec((HID, 3 * K), lambda i: (0, 0)),
            pl.BlockSpec((1, 3 * K), lambda i: (0, 0)),
        ],
        out_specs=[
            pl.BlockSpec((8, NC, r), lambda i: (0, 0, i)),
            pl.BlockSpec((8, NC, r), lambda i: (0, 0, i)),
        ],
        out_shape=[
            jax.ShapeDtypeStruct((8, NC, B), jnp.int32),
            jax.ShapeDtypeStruct((8, NC, B), jnp.float32),
        ],
    )(*rows, ts_row, mbt, mw1t, mb1c, mw2, mb2r)


def _make_gather_blend(p_total, chunk):
    pw = p_total // NW
    nch = pw // chunk
    assert pw % chunk == 0 and chunk % 128 == 0 and nch % 2 == 0
    mesh = plsc.VectorSubcoreMesh(core_axis_name="c", subcore_axis_name="s",
                                  num_cores=SC_CORES, num_subcores=SC_SUBCORES)

    @functools.partial(
        pl.kernel,
        out_type=jax.ShapeDtypeStruct((p_total, VD), jnp.float32),
        mesh=mesh,
        scratch_types=[
            pltpu.VMEM((8 * chunk,), jnp.int32),
            pltpu.VMEM((8 * chunk,), jnp.int32),
            pltpu.VMEM((8 * chunk, VD), jnp.float32),
            pltpu.VMEM((8 * chunk, VD), jnp.float32),
            pltpu.VMEM((8 * chunk,), jnp.float32),
            pltpu.VMEM((8 * chunk,), jnp.float32),
            pltpu.VMEM((chunk, VD), jnp.float32),
            pltpu.SemaphoreType.DMA,
            pltpu.SemaphoreType.DMA,
        ],
        compiler_params=pltpu.CompilerParams(use_tc_tiling_on_sc=False,
                                             needs_layout_passes=False),
    )
    def k(table, idx2d, w2d, out_hbm, idx0, idx1, rows0, rows1, w0, w1,
          out_v, sem0, sem1):
        cid = lax.axis_index("c")
        sid = lax.axis_index("s")
        wid = sid * SC_CORES + cid
        idx_b = (idx0, idx1)
        rows_b = (rows0, rows1)
        w_b = (w0, w1)
        sem_b = (sem0, sem1)

        def stage(ch, buf):
            base = wid * pw + ch * chunk
            for j in range(8):
                pltpu.sync_copy(idx2d.at[j, pl.ds(base, chunk)],
                                idx_b[buf].at[pl.ds(j * chunk, chunk)])
                pltpu.sync_copy(w2d.at[j, pl.ds(base, chunk)],
                                w_b[buf].at[pl.ds(j * chunk, chunk)])
            pltpu.async_copy(table.at[idx_b[buf]], rows_b[buf], sem_b[buf])

        def blend(ch, buf):
            base = wid * pw + ch * chunk
            pltpu.make_async_copy(
                table.at[idx_b[buf]], rows_b[buf], sem_b[buf]).wait()

            def point_body(p, c2):
                acc = jnp.zeros((VD,), jnp.float32)
                for j in range(8):
                    wsp = plsc.load_gather(
                        w_b[buf], [jnp.broadcast_to(j * chunk + p, (VD,))])
                    row = rows_b[buf][j * chunk + p, :]
                    acc = acc + wsp * row
                out_v[p, :] = acc
                return c2

            lax.fori_loop(0, chunk, point_body, 0)
            pltpu.sync_copy(out_v, out_hbm.at[pl.ds(base, chunk), :])

        stage(0, 0)

        def pair_body(g, carry):
            ch0 = g * 2
            stage(ch0 + 1, 1)
            blend(ch0, 0)

            @pl.when(ch0 + 2 < nch)
            def _():
                stage(ch0 + 2, 0)

            blend(ch0 + 1, 1)
            return carry

        lax.fori_loop(0, nch // 2, pair_body, 0)

    return k


def _tc2_body(feat0_ref, ox, oy, oz, dx, dy, dz, ts_ref, mbt_ref, mw1t_ref,
              mb1c_ref, mw2_ref, mb2r_ref, u_ref, dv2_ref, idx_ref, w_ref):
    r = ox.shape[1]
    feat0 = feat0_ref[...]
    dens = jnp.maximum(feat0, 0.0) + jnp.log1p(jnp.exp(-jnp.abs(feat0)))

    dvv = NEAR + STEP * lax.broadcasted_iota(
        jnp.int32, (NC, 1), 0).astype(jnp.float32)
    dist = jnp.concatenate(
        [dvv[1:] - dvv[:-1], jnp.full((1, 1), 1e10, jnp.float32)], axis=0)
    e = jnp.exp(-dens * dist)
    alpha = 1.0 - e
    vte = e + 1e-10
    logv = jnp.log(vte)
    mstrict = (lax.broadcasted_iota(jnp.int32, (NC, NC), 1)
               < lax.broadcasted_iota(jnp.int32, (NC, NC), 0)).astype(jnp.float32)
    tt = jnp.exp(lax.dot_general(mstrict, logv, (((1,), (0,)), ((), ())),
                                 precision=HIGH))
    w = tt * alpha

    m = NC - 2
    wp = w[1:NC - 1, :] + 1e-5
    ones = jnp.ones((1, m), jnp.float32)
    wsum = lax.dot_general(ones, wp, (((1,), (0,)), ((), ())), precision=HIGH)
    pdf = wp / wsum
    minc = (lax.broadcasted_iota(jnp.int32, (m, m), 1)
            <= lax.broadcasted_iota(jnp.int32, (m, m), 0)).astype(jnp.float32)
    cdf = lax.dot_general(minc, pdf, (((1,), (0,)), ((), ())), precision=HIGH)
    c62 = jnp.concatenate([jnp.zeros((1, r), jnp.float32), cdf[:m - 1, :]],
                          axis=0)
    u = u_ref[...]
    ge = u[None, :, :] >= c62[:, None, :]
    gef = ge.astype(jnp.float32)
    above_f = _sum0(gef)
    cdf_b = _max0(gef * c62[:, None, :])
    cdf_a3 = jnp.where(ge, 2.0, jnp.broadcast_to(c62[:, None, :], ge.shape))
    cdf_a = jnp.minimum(_min0(cdf_a3), cdf[m - 1:m, :])
    denom = cdf_a - cdf_b
    denom = jnp.where(denom < 1e-5, 1.0, denom)
    t = (u - cdf_b) / denom
    bins_b = NEAR + STEP * (above_f - 0.5)
    samples = bins_b + t * STEP

    x = jnp.concatenate([jnp.broadcast_to(dvv, (NC, r)), samples], axis=0)
    sub = lax.broadcasted_iota(jnp.int32, (N2, 1), 0)
    k = 2
    while k <= N2:
        asc = (sub & k) == 0
        j = k // 2
        while j >= 1:
            upper = (sub & j) != 0
            partner = jnp.where(upper, jnp.roll(x, j, axis=0),
                                jnp.roll(x, -j, axis=0))
            takemin = (~upper) == asc
            x = jnp.where(takemin, jnp.minimum(x, partner),
                          jnp.maximum(x, partner))
            j //= 2
        k *= 2
    dv2_ref[...] = x
    _prep_write((ox, oy, oz), (dx, dy, dz), ts_ref, mbt_ref, mw1t_ref,
                mb1c_ref, mw2_ref, mb2r_ref, x, idx_ref, w_ref)


def _fine_prep(feat0c, rows, ts_row, mbt, mw1t, mb1c, mw2, mb2r, u_t):
    r = 256
    grid = B // r
    return pl.pallas_call(
        _tc2_body,
        grid=(grid,),
        in_specs=[pl.BlockSpec((NC, r), lambda i: (0, i))]
        + [_row_spec(r)] * 6 + [
            pl.BlockSpec((1, r), lambda i: (0, i)),
            pl.BlockSpec((K, T), lambda i: (0, 0)),
            pl.BlockSpec((HID, 3), lambda i: (0, 0)),
            pl.BlockSpec((HID, 1), lambda i: (0, 0)),
            pl.BlockSpec((HID, 3 * K), lambda i: (0, 0)),
            pl.BlockSpec((1, 3 * K), lambda i: (0, 0)),
            pl.BlockSpec((NF, r), lambda i: (0, i)),
        ],
        out_specs=[
            pl.BlockSpec((N2, r), lambda i: (0, i)),
            pl.BlockSpec((8, N2, r), lambda i: (0, 0, i)),
            pl.BlockSpec((8, N2, r), lambda i: (0, 0, i)),
        ],
        out_shape=[
            jax.ShapeDtypeStruct((N2, B), jnp.float32),
            jax.ShapeDtypeStruct((8, N2, B), jnp.int32),
            jax.ShapeDtypeStruct((8, N2, B), jnp.float32),
        ],
    )(feat0c, *rows, ts_row, mbt, mw1t, mb1c, mw2, mb2r, u_t)


def _tc3a_body(feat_ref, cw1_ref, cb1_ref, cw2_ref, cb2_ref, out_ref):
    f2 = feat_ref[...]
    h = jnp.maximum(jnp.dot(f2, cw1_ref[...]) + cb1_ref[...], 0.0)
    out_ref[...] = jnp.dot(h, cw2_ref[...]) + cb2_ref[...]


def _color_mlp(featf, cw1, cb1, cw2, cb2):
    rp = 8192
    grid = (B * N2) // rp
    return pl.pallas_call(
        _tc3a_body,
        grid=(grid,),
        in_specs=[
            pl.BlockSpec((rp, VD), lambda i: (i, 0)),
            pl.BlockSpec((VD, HID), lambda i: (0, 0)),
            pl.BlockSpec((1, HID), lambda i: (0, 0)),
            pl.BlockSpec((HID, 3), lambda i: (0, 0)),
            pl.BlockSpec((1, 3), lambda i: (0, 0)),
        ],
        out_specs=pl.BlockSpec((rp, 3), lambda i: (i, 0)),
        out_shape=jax.ShapeDtypeStruct((B * N2, 3), jnp.float32),
    )(featf, cw1, cb1, cw2, cb2)


def _tc3b_body(feat0_ref, lx_ref, ly_ref, lz_ref, dv2_ref, out_ref):
    feat0 = feat0_ref[...]
    dens = jnp.maximum(feat0, 0.0) + jnp.log1p(jnp.exp(-jnp.abs(feat0)))
    dv2 = dv2_ref[...]
    r = dv2.shape[1]
    dist = jnp.concatenate(
        [dv2[1:, :] - dv2[:-1, :], jnp.full((1, r), 1e10, jnp.float32)], axis=0)
    e = jnp.exp(-dens * dist)
    alpha = 1.0 - e
    vte = e + 1e-10
    logv = jnp.log(vte)
    mstrict = (lax.broadcasted_iota(jnp.int32, (N2, N2), 1)
               < lax.broadcasted_iota(jnp.int32, (N2, N2), 0)).astype(jnp.float32)
    tt = jnp.exp(lax.dot_general(mstrict, logv, (((1,), (0,)), ((), ())),
                                 precision=HIGH))
    w = tt * alpha
    ones = jnp.ones((1, N2), jnp.float32)
    chans = [lax.dot_general(ones, w * jax.nn.sigmoid(c_ref[...]),
                             (((1,), (0,)), ((), ())), precision=HIGH)
             for c_ref in (lx_ref, ly_ref, lz_ref)]
    out_ref[...] = jnp.concatenate(chans, axis=0)


def _final_render(feat0f, lx, ly, lz, dv2):
    r = 512
    grid = B // r
    return pl.pallas_call(
        _tc3b_body,
        grid=(grid,),
        in_specs=[pl.BlockSpec((N2, r), lambda i: (0, i))] * 5,
        out_specs=pl.BlockSpec((3, r), lambda i: (0, i)),
        out_shape=jax.ShapeDtypeStruct((3, B), jnp.float32),
    )(feat0f, lx, ly, lz, dv2)


def kernel(ray_origin, ray_direction, time_step, motion_basis, mw1, mb1, mw2,
           mb2, voxel_grid, cw1, cb1, cw2, cb2):
    ot = ray_origin.T
    dt = ray_direction.T
    ts_row = time_step.astype(jnp.int32).reshape(1, B)
    rows = (ot[0:1], ot[1:2], ot[2:3], dt[0:1], dt[1:2], dt[2:3])
    mbt = motion_basis.T
    mw1t = mw1.T
    mb1c = mb1.reshape(HID, 1)
    mb2r = mb2.reshape(1, 3 * K)
    cb1r = cb1.reshape(1, HID)
    table = voxel_grid.reshape(V * V * V, VD)
    u_t = jax.random.uniform(jax.random.key(1), (B, NF), dtype=jnp.float32).T

    idxc, wc = _coarse_prep(rows, ts_row, mbt, mw1t, mb1c, mw2, mb2r)
    pc = B * NC
    featc = _make_gather_blend(pc, 256)(
        table, idxc.reshape(8, pc), wc.reshape(8, pc))
    feat0c = featc[:, 0].reshape(NC, B)
    dv2, idxf, wf = _fine_prep(feat0c, rows, ts_row, mbt, mw1t, mb1c, mw2,
                               mb2r, u_t)
    pf = B * N2
    featf = _make_gather_blend(pf, 256)(
        table, idxf.reshape(8, pf), wf.reshape(8, pf))
    logits = _color_mlp(featf, cw1, cb1r, cw2, cb2.reshape(1, 3))
    l3 = logits.reshape(N2, B, 3)
    feat0f = featf[:, 0].reshape(N2, B)
    colors_t = _final_render(feat0f, l3[:, :, 0], l3[:, :, 1], l3[:, :, 2],
                             dv2)
    return colors_t.T


# speedup vs baseline: 1.2502x; 1.0489x over previous
import functools

import jax
import jax.numpy as jnp
from jax import lax
from jax.experimental import pallas as pl
from jax.experimental.pallas import tpu as pltpu
from jax.experimental.pallas import tpu_sc as plsc

B = 4096
NC = 64
NF = 64
N2 = NC + NF
NEAR = 2.0
FAR = 6.0
STEP = (FAR - NEAR) / (NC - 1)
K = 8
V = 128
MINB = -4.0
MAXB = 4.0
VD = 16
HID = 64
T = 100
GSCALE = (V - 1) / (MAXB - MINB)

SC_CORES = 2
SC_SUBCORES = 16
NW = SC_CORES * SC_SUBCORES

HIGH = lax.Precision.HIGHEST


def _sum0(x):
    while x.shape[0] > 1:
        s = x.shape[0]
        h = s // 2
        y = x[:h] + x[h:2 * h]
        x = y if s % 2 == 0 else jnp.concatenate([y, x[2 * h:]], axis=0)
    return x[0]


def _max0(x):
    while x.shape[0] > 1:
        s = x.shape[0]
        h = s // 2
        y = jnp.maximum(x[:h], x[h:2 * h])
        x = y if s % 2 == 0 else jnp.concatenate([y, x[2 * h:]], axis=0)
    return x[0]


def _min0(x):
    while x.shape[0] > 1:
        s = x.shape[0]
        h = s // 2
        y = jnp.minimum(x[:h], x[h:2 * h])
        x = y if s % 2 == 0 else jnp.concatenate([y, x[2 * h:]], axis=0)
    return x[0]


def _bt_t(ts_row, mbt_ref):
    r = ts_row.shape[1]
    onehot = (lax.broadcasted_iota(jnp.int32, (T, r), 0)
              == ts_row).astype(jnp.float32)
    return lax.dot_general(mbt_ref[...], onehot, (((1,), (0,)), ((), ())),
                           precision=HIGH)


def _warp_corners_t(oc, dc, depths_t, bt_t, mw1t_ref, mb1c_ref, mw2_ref,
                    mb2r_ref):
    w1 = mw1t_ref[...]
    w2 = mw2_ref[...]
    mb2r = mb2r_ref[...]
    w1c = [w1[:, c:c + 1] for c in range(3)]
    a2 = (w1c[0] * oc[0] + w1c[1] * oc[1] + w1c[2] * oc[2]
          + mb1c_ref[...])
    b2 = w1c[0] * dc[0] + w1c[1] * dc[1] + w1c[2] * dc[2]
    h = jnp.tanh(a2[:, None, :] + b2[:, None, :] * depths_t[None, :, :])

    gs = []
    for c in range(3):
        vc = w2[:, c:c + 1] * bt_t[0:1, :]
        mbc = mb2r[0:1, c:c + 1] * bt_t[0:1, :]
        for k in range(1, K):
            col = 3 * k + c
            vc = vc + w2[:, col:col + 1] * bt_t[k:k + 1, :]
            mbc = mbc + mb2r[0:1, col:col + 1] * bt_t[k:k + 1, :]
        disp = _sum0(h * vc[:, None, :]) + mbc
        wpos = oc[c] + dc[c] * depths_t + disp
        g = (wpos - MINB) * GSCALE
        gs.append(jnp.clip(g, 0.0, V - 1 - 1e-6))

    g0f = [jnp.floor(g) for g in gs]
    f = [g - g0 for g, g0 in zip(gs, g0f)]
    gi = [g0.astype(jnp.int32) for g0 in g0f]
    hi = [jnp.minimum(g + 1, V - 1) for g in gi]
    gx, gy, gz = gi
    hx, hy, hz = hi
    idx8 = [(a_ * V + b_) * V + c_ for a_, b_, c_ in
            ((gx, gy, gz), (gx, gy, hz), (gx, hy, gz), (gx, hy, hz),
             (hx, gy, gz), (hx, gy, hz), (hx, hy, gz), (hx, hy, hz))]
    return idx8, f


def _prep_write(ors, drs, ts_ref, mbt_ref, mw1t_ref, mb1c_ref, mw2_ref,
                mb2r_ref, depths_t, idx_ref, w_ref):
    bt_t = _bt_t(ts_ref[...], mbt_ref)
    oc = [o_ref[...] for o_ref in ors]
    dc = [d_ref[...] for d_ref in drs]
    idx8, f3 = _warp_corners_t(oc, dc, depths_t, bt_t, mw1t_ref, mb1c_ref,
                               mw2_ref, mb2r_ref)
    for j in range(8):
        idx_ref[j] = idx8[j]
    for c in range(3):
        w_ref[c] = f3[c]


def _tc1_body(ox, oy, oz, dx, dy, dz, ts_ref, mbt_ref, mw1t_ref, mb1c_ref,
              mw2_ref, mb2r_ref, idx_ref, w_ref):
    r = ox.shape[1]
    depths_t = NEAR + STEP * lax.broadcasted_iota(
        jnp.int32, (NC, r), 0).astype(jnp.float32)
    _prep_write((ox, oy, oz), (dx, dy, dz), ts_ref, mbt_ref, mw1t_ref,
                mb1c_ref, mw2_ref, mb2r_ref, depths_t, idx_ref, w_ref)


def _row_spec(r):
    return pl.BlockSpec((1, r), lambda i: (0, i))


def _coarse_prep(rows, ts_row, mbt, mw1t, mb1c, mw2, mb2r):
    r = 512
    grid = B // r
    return pl.pallas_call(
        _tc1_body,
        grid=(grid,),
        in_specs=[_row_spec(r)] * 6 + [
            pl.BlockSpec((1, r), lambda i: (0, i)),
            pl.BlockSpec((K, T), lambda i: (0, 0)),
            pl.BlockSpec((HID, 3), lambda i: (0, 0)),
            pl.BlockSpec((HID, 1), lambda i: (0, 0)),
            pl.BlockSpec((HID, 3 * K), lambda i: (0, 0)),
            pl.BlockSpec((1, 3 * K), lambda i: (0, 0)),
        ],
        out_specs=[
            pl.BlockSpec((8, NC, r), lambda i: (0, 0, i)),
            pl.BlockSpec((3, NC, r), lambda i: (0, 0, i)),
        ],
        out_shape=[
            jax.ShapeDtypeStruct((8, NC, B), jnp.int32),
            jax.ShapeDtypeStruct((3, NC, B), jnp.float32),
        ],
    )(*rows, ts_row, mbt, mw1t, mb1c, mw2, mb2r)


def _make_gather_blend(p_total, chunk):
    pw = p_total // NW
    nch = pw // chunk
    assert pw % chunk == 0 and chunk % 128 == 0 and nch % 2 == 0
    mesh = plsc.VectorSubcoreMesh(core_axis_name="c", subcore_axis_name="s",
                                  num_cores=SC_CORES, num_subcores=SC_SUBCORES)

    @functools.partial(
        pl.kernel,
        out_type=jax.ShapeDtypeStruct((p_total, VD), jnp.float32),
        mesh=mesh,
        scratch_types=[
            pltpu.VMEM((8 * chunk,), jnp.int32),
            pltpu.VMEM((8 * chunk,), jnp.int32),
            pltpu.VMEM((8 * chunk, VD), jnp.float32),
            pltpu.VMEM((8 * chunk, VD), jnp.float32),
            pltpu.VMEM((3 * chunk,), jnp.float32),
            pltpu.VMEM((3 * chunk,), jnp.float32),
            pltpu.VMEM((chunk, VD), jnp.float32),
            pltpu.SemaphoreType.DMA,
            pltpu.SemaphoreType.DMA,
        ],
        compiler_params=pltpu.CompilerParams(use_tc_tiling_on_sc=False,
                                             needs_layout_passes=False),
    )
    def k(table, idx2d, w2d, out_hbm, idx0, idx1, rows0, rows1, w0, w1,
          out_v, sem0, sem1):
        cid = lax.axis_index("c")
        sid = lax.axis_index("s")
        wid = sid * SC_CORES + cid
        idx_b = (idx0, idx1)
        rows_b = (rows0, rows1)
        w_b = (w0, w1)
        sem_b = (sem0, sem1)

        def stage(ch, buf):
            base = wid * pw + ch * chunk
            for j in range(8):
                pltpu.sync_copy(idx2d.at[j, pl.ds(base, chunk)],
                                idx_b[buf].at[pl.ds(j * chunk, chunk)])
            for c in range(3):
                pltpu.sync_copy(w2d.at[c, pl.ds(base, chunk)],
                                w_b[buf].at[pl.ds(c * chunk, chunk)])
            pltpu.async_copy(table.at[idx_b[buf]], rows_b[buf], sem_b[buf])

        def blend(ch, buf):
            base = wid * pw + ch * chunk
            pltpu.make_async_copy(
                table.at[idx_b[buf]], rows_b[buf], sem_b[buf]).wait()

            def point_body(p, c2):
                fr = [plsc.load_gather(
                    w_b[buf], [jnp.broadcast_to(c * chunk + p, (VD,))])
                    for c in range(3)]
                fx, fy, fz = fr
                ex, ey, ez = 1.0 - fx, 1.0 - fy, 1.0 - fz
                w8 = (ex * ey * ez, ex * ey * fz, ex * fy * ez, ex * fy * fz,
                      fx * ey * ez, fx * ey * fz, fx * fy * ez, fx * fy * fz)
                acc = jnp.zeros((VD,), jnp.float32)
                for j in range(8):
                    acc = acc + w8[j] * rows_b[buf][j * chunk + p, :]
                out_v[p, :] = acc
                return c2

            lax.fori_loop(0, chunk, point_body, 0)
            pltpu.sync_copy(out_v, out_hbm.at[pl.ds(base, chunk), :])

        stage(0, 0)

        def pair_body(g, carry):
            ch0 = g * 2
            stage(ch0 + 1, 1)
            blend(ch0, 0)

            @pl.when(ch0 + 2 < nch)
            def _():
                stage(ch0 + 2, 0)

            blend(ch0 + 1, 1)
            return carry

        lax.fori_loop(0, nch // 2, pair_body, 0)

    return k


def _tc2_body(feat0_ref, ox, oy, oz, dx, dy, dz, ts_ref, mbt_ref, mw1t_ref,
              mb1c_ref, mw2_ref, mb2r_ref, u_ref, dv2_ref, idx_ref, w_ref):
    r = ox.shape[1]
    feat0 = feat0_ref[...]
    dens = jnp.maximum(feat0, 0.0) + jnp.log1p(jnp.exp(-jnp.abs(feat0)))

    dvv = NEAR + STEP * lax.broadcasted_iota(
        jnp.int32, (NC, 1), 0).astype(jnp.float32)
    dist = jnp.concatenate(
        [dvv[1:] - dvv[:-1], jnp.full((1, 1), 1e10, jnp.float32)], axis=0)
    e = jnp.exp(-dens * dist)
    alpha = 1.0 - e
    vte = e + 1e-10
    logv = jnp.log(vte)
    mstrict = (lax.broadcasted_iota(jnp.int32, (NC, NC), 1)
               < lax.broadcasted_iota(jnp.int32, (NC, NC), 0)).astype(jnp.float32)
    tt = jnp.exp(lax.dot_general(mstrict, logv, (((1,), (0,)), ((), ())),
                                 precision=HIGH))
    w = tt * alpha

    m = NC - 2
    wp = w[1:NC - 1, :] + 1e-5
    ones = jnp.ones((1, m), jnp.float32)
    wsum = lax.dot_general(ones, wp, (((1,), (0,)), ((), ())), precision=HIGH)
    pdf = wp / wsum
    minc = (lax.broadcasted_iota(jnp.int32, (m, m), 1)
            <= lax.broadcasted_iota(jnp.int32, (m, m), 0)).astype(jnp.float32)
    cdf = lax.dot_general(minc, pdf, (((1,), (0,)), ((), ())), precision=HIGH)
    c62 = jnp.concatenate([jnp.zeros((1, r), jnp.float32), cdf[:m - 1, :]],
                          axis=0)
    u = u_ref[...]
    ge = u[None, :, :] >= c62[:, None, :]
    gef = ge.astype(jnp.float32)
    above_f = _sum0(gef)
    cdf_b = _max0(gef * c62[:, None, :])
    cdf_a3 = jnp.where(ge, 2.0, jnp.broadcast_to(c62[:, None, :], ge.shape))
    cdf_a = jnp.minimum(_min0(cdf_a3), cdf[m - 1:m, :])
    denom = cdf_a - cdf_b
    denom = jnp.where(denom < 1e-5, 1.0, denom)
    t = (u - cdf_b) / denom
    bins_b = NEAR + STEP * (above_f - 0.5)
    samples = bins_b + t * STEP

    x = jnp.concatenate([jnp.broadcast_to(dvv, (NC, r)), samples], axis=0)
    sub = lax.broadcasted_iota(jnp.int32, (N2, 1), 0)
    k = 2
    while k <= N2:
        asc = (sub & k) == 0
        j = k // 2
        while j >= 1:
            upper = (sub & j) != 0
            partner = jnp.where(upper, jnp.roll(x, j, axis=0),
                                jnp.roll(x, -j, axis=0))
            takemin = (~upper) == asc
            x = jnp.where(takemin, jnp.minimum(x, partner),
                          jnp.maximum(x, partner))
            j //= 2
        k *= 2
    dv2_ref[...] = x
    _prep_write((ox, oy, oz), (dx, dy, dz), ts_ref, mbt_ref, mw1t_ref,
                mb1c_ref, mw2_ref, mb2r_ref, x, idx_ref, w_ref)


def _fine_prep(feat0c, rows, ts_row, mbt, mw1t, mb1c, mw2, mb2r, u_t):
    r = 256
    grid = B // r
    return pl.pallas_call(
        _tc2_body,
        grid=(grid,),
        in_specs=[pl.BlockSpec((NC, r), lambda i: (0, i))]
        + [_row_spec(r)] * 6 + [
            pl.BlockSpec((1, r), lambda i: (0, i)),
            pl.BlockSpec((K, T), lambda i: (0, 0)),
            pl.BlockSpec((HID, 3), lambda i: (0, 0)),
            pl.BlockSpec((HID, 1), lambda i: (0, 0)),
            pl.BlockSpec((HID, 3 * K), lambda i: (0, 0)),
            pl.BlockSpec((1, 3 * K), lambda i: (0, 0)),
            pl.BlockSpec((NF, r), lambda i: (0, i)),
        ],
        out_specs=[
            pl.BlockSpec((N2, r), lambda i: (0, i)),
            pl.BlockSpec((8, N2, r), lambda i: (0, 0, i)),
            pl.BlockSpec((3, N2, r), lambda i: (0, 0, i)),
        ],
        out_shape=[
            jax.ShapeDtypeStruct((N2, B), jnp.float32),
            jax.ShapeDtypeStruct((8, N2, B), jnp.int32),
            jax.ShapeDtypeStruct((3, N2, B), jnp.float32),
        ],
    )(feat0c, *rows, ts_row, mbt, mw1t, mb1c, mw2, mb2r, u_t)


def _tc3a_body(feat_ref, cw1_ref, cb1_ref, cw2_ref, cb2_ref, out_ref):
    f2 = feat_ref[...]
    h = jnp.maximum(jnp.dot(f2, cw1_ref[...]) + cb1_ref[...], 0.0)
    out_ref[...] = jnp.dot(h, cw2_ref[...]) + cb2_ref[...]


def _color_mlp(featf, cw1, cb1, cw2, cb2):
    rp = 8192
    grid = (B * N2) // rp
    return pl.pallas_call(
        _tc3a_body,
        grid=(grid,),
        in_specs=[
            pl.BlockSpec((rp, VD), lambda i: (i, 0)),
            pl.BlockSpec((VD, HID), lambda i: (0, 0)),
            pl.BlockSpec((1, HID), lambda i: (0, 0)),
            pl.BlockSpec((HID, 3), lambda i: (0, 0)),
            pl.BlockSpec((1, 3), lambda i: (0, 0)),
        ],
        out_specs=pl.BlockSpec((rp, 3), lambda i: (i, 0)),
        out_shape=jax.ShapeDtypeStruct((B * N2, 3), jnp.float32),
    )(featf, cw1, cb1, cw2, cb2)


def _tc3b_body(feat0_ref, lx_ref, ly_ref, lz_ref, dv2_ref, out_ref):
    feat0 = feat0_ref[...]
    dens = jnp.maximum(feat0, 0.0) + jnp.log1p(jnp.exp(-jnp.abs(feat0)))
    dv2 = dv2_ref[...]
    r = dv2.shape[1]
    dist = jnp.concatenate(
        [dv2[1:, :] - dv2[:-1, :], jnp.full((1, r), 1e10, jnp.float32)], axis=0)
    e = jnp.exp(-dens * dist)
    alpha = 1.0 - e
    vte = e + 1e-10
    logv = jnp.log(vte)
    mstrict = (lax.broadcasted_iota(jnp.int32, (N2, N2), 1)
               < lax.broadcasted_iota(jnp.int32, (N2, N2), 0)).astype(jnp.float32)
    tt = jnp.exp(lax.dot_general(mstrict, logv, (((1,), (0,)), ((), ())),
                                 precision=HIGH))
    w = tt * alpha
    ones = jnp.ones((1, N2), jnp.float32)
    chans = [lax.dot_general(ones, w * jax.nn.sigmoid(c_ref[...]),
                             (((1,), (0,)), ((), ())), precision=HIGH)
             for c_ref in (lx_ref, ly_ref, lz_ref)]
    out_ref[...] = jnp.concatenate(chans, axis=0)


def _final_render(feat0f, lx, ly, lz, dv2):
    r = 512
    grid = B // r
    return pl.pallas_call(
        _tc3b_body,
        grid=(grid,),
        in_specs=[pl.BlockSpec((N2, r), lambda i: (0, i))] * 5,
        out_specs=pl.BlockSpec((3, r), lambda i: (0, i)),
        out_shape=jax.ShapeDtypeStruct((3, B), jnp.float32),
    )(feat0f, lx, ly, lz, dv2)


def kernel(ray_origin, ray_direction, time_step, motion_basis, mw1, mb1, mw2,
           mb2, voxel_grid, cw1, cb1, cw2, cb2):
    ot = ray_origin.T
    dt = ray_direction.T
    ts_row = time_step.astype(jnp.int32).reshape(1, B)
    rows = (ot[0:1], ot[1:2], ot[2:3], dt[0:1], dt[1:2], dt[2:3])
    mbt = motion_basis.T
    mw1t = mw1.T
    mb1c = mb1.reshape(HID, 1)
    mb2r = mb2.reshape(1, 3 * K)
    cb1r = cb1.reshape(1, HID)
    table = voxel_grid.reshape(V * V * V, VD)
    u_t = jax.random.uniform(jax.random.key(1), (B, NF), dtype=jnp.float32).T

    idxc, wc = _coarse_prep(rows, ts_row, mbt, mw1t, mb1c, mw2, mb2r)
    pc = B * NC
    featc = _make_gather_blend(pc, 256)(
        table, idxc.reshape(8, pc), wc.reshape(3, pc))
    feat0c = featc[:, 0].reshape(NC, B)
    dv2, idxf, wf = _fine_prep(feat0c, rows, ts_row, mbt, mw1t, mb1c, mw2,
                               mb2r, u_t)
    pf = B * N2
    featf = _make_gather_blend(pf, 256)(
        table, idxf.reshape(8, pf), wf.reshape(3, pf))
    logits = _color_mlp(featf, cw1, cb1r, cw2, cb2.reshape(1, 3))
    l3 = logits.reshape(N2, B, 3)
    feat0f = featf[:, 0].reshape(N2, B)
    colors_t = _final_render(feat0f, l3[:, :, 0], l3[:, :, 1], l3[:, :, 2],
                             dv2)
    return colors_t.T
